# baseline (speedup 1.0000x reference)
import functools

import jax
import jax.numpy as jnp
from jax import lax
from jax.experimental import pallas as pl
from jax.experimental.pallas import tpu as pltpu
from jax.experimental.pallas import tpu_sc as plsc

LANES = 16
N_BUF = 4
CH = 32


@functools.lru_cache(maxsize=None)
def _build_sc_embed(S, B, D, n_workers):
    N = S * B
    per_w = N // n_workers
    n_ch = per_w // CH
    pos_ch = CH // B
    per_pos = per_w // B

    mesh = plsc.VectorSubcoreMesh(core_axis_name="c", subcore_axis_name="s")

    @functools.partial(
        pl.kernel,
        mesh=mesh,
        out_type=jax.ShapeDtypeStruct((S, B, D), jnp.float32),
        scratch_types=[
            pltpu.VMEM((n_ch, CH), jnp.int32),
            pltpu.VMEM((N_BUF, CH, D), jnp.float32),
            pltpu.VMEM((N_BUF, pos_ch, D), jnp.float32),
            pltpu.SemaphoreType.DMA((N_BUF,)),
            pltpu.SemaphoreType.DMA((N_BUF,)),
            pltpu.SemaphoreType.DMA((N_BUF,)),
        ],
    )
    def sc_embed(x_hbm, tok_hbm, pos_hbm, out_hbm, idx_v, tok_v, pos_v,
                 gsem, psem, osem):
        out2 = out_hbm.reshape(S * B, D)
        x2 = x_hbm
        num_c = lax.axis_size("c")
        wid = lax.axis_index("s") * num_c + lax.axis_index("c")
        row0 = wid * per_w
        pos0 = wid * per_pos

        pltpu.sync_copy(x2.at[wid], idx_v)

        H = 8
        NH = CH // H

        def start_in(g, p):
            for h in range(NH):
                pltpu.async_copy(tok_hbm.at[idx_v.at[g, pl.ds(h * H, H)]],
                                 tok_v.at[p, pl.ds(h * H, H)], gsem.at[p])
            pltpu.async_copy(
                pos_hbm.at[pl.ds(pos0 + g * pos_ch, pos_ch)], pos_v.at[p],
                psem.at[p])

        def wait_in(p):
            for h in range(NH):
                pltpu.make_async_copy(
                    tok_hbm.at[idx_v.at[0, pl.ds(0, H)]],
                    tok_v.at[p, pl.ds(h * H, H)], gsem.at[p]).wait()
            pltpu.make_async_copy(
                pos_hbm.at[pl.ds(0, pos_ch)], pos_v.at[p], psem.at[p]).wait()

        def start_out(g, p):
            for h in range(NH):
                pltpu.async_copy(tok_v.at[p, pl.ds(h * H, H)],
                                 out2.at[pl.ds(row0 + g * CH + h * H, H)],
                                 osem.at[p])

        def wait_out(p):
            for h in range(NH):
                pltpu.make_async_copy(
                    tok_v.at[p, pl.ds(h * H, H)], out2.at[pl.ds(0, H)],
                    osem.at[p]).wait()

        for g in range(N_BUF - 1):
            start_in(g, g)

        def chunk(g, carry):
            p = lax.rem(g, N_BUF)
            wait_in(p)

            def add_col(d, c):
                sl = pl.ds(d * LANES, LANES)
                pvs = [pos_v[p, r, sl] for r in range(pos_ch)]
                for r in range(pos_ch):
                    for b in range(B):
                        plsc.addupdate(tok_v.at[p, r * B + b, sl], pvs[r])
                return c

            lax.fori_loop(0, D // LANES, add_col, 0)

            start_out(g, p)

            @pl.when(g + N_BUF - 1 < n_ch)
            def _():
                q = lax.rem(g + N_BUF - 1, N_BUF)

                @pl.when(g >= 1)
                def _():
                    wait_out(q)

                start_in(g + N_BUF - 1, q)

            return carry

        lax.fori_loop(0, n_ch, chunk, 0)

        for i in range(N_BUF):
            wait_out((n_ch - N_BUF + i) % N_BUF)

    return sc_embed


def kernel(x, token_table, pos_table):
    S, B = x.shape
    D = token_table.shape[1]
    info = plsc.get_sparse_core_info()
    n_workers = info.num_cores * info.num_subcores
    xf = x.reshape(n_workers, -1, CH).astype(jnp.int32)
    return _build_sc_embed(S, B, D, n_workers)(xf, token_table, pos_table)


# speedup vs baseline: 1.0226x; 1.0226x over previous
import functools

import jax
import jax.numpy as jnp
from jax import lax
from jax.experimental import pallas as pl
from jax.experimental.pallas import tpu as pltpu
from jax.experimental.pallas import tpu_sc as plsc

LANES = 16
N_BUF = 8
CH = 16


@functools.lru_cache(maxsize=None)
def _build_sc_embed(S, B, D, n_workers):
    N = S * B
    per_w = N // n_workers
    n_ch = per_w // CH
    pos_ch = CH // B
    per_pos = per_w // B

    mesh = plsc.VectorSubcoreMesh(core_axis_name="c", subcore_axis_name="s")

    @functools.partial(
        pl.kernel,
        mesh=mesh,
        out_type=jax.ShapeDtypeStruct((S, B, D), jnp.float32),
        scratch_types=[
            pltpu.VMEM((n_ch, CH), jnp.int32),
            pltpu.VMEM((N_BUF, CH, D), jnp.float32),
            pltpu.VMEM((N_BUF, pos_ch, D), jnp.float32),
            pltpu.SemaphoreType.DMA((N_BUF,)),
            pltpu.SemaphoreType.DMA((N_BUF,)),
            pltpu.SemaphoreType.DMA((N_BUF,)),
        ],
    )
    def sc_embed(x_hbm, tok_hbm, pos_hbm, out_hbm, idx_v, tok_v, pos_v,
                 gsem, psem, osem):
        out2 = out_hbm.reshape(S * B, D)
        x2 = x_hbm
        num_c = lax.axis_size("c")
        wid = lax.axis_index("s") * num_c + lax.axis_index("c")
        row0 = wid * per_w
        pos0 = wid * per_pos

        pltpu.sync_copy(x2.at[wid], idx_v)

        H = CH // 2

        def start_in(g, p):
            pltpu.async_copy(tok_hbm.at[idx_v.at[g, pl.ds(0, H)]],
                             tok_v.at[p, pl.ds(0, H)], gsem.at[p])
            pltpu.async_copy(tok_hbm.at[idx_v.at[g, pl.ds(H, H)]],
                             tok_v.at[p, pl.ds(H, H)], gsem.at[p])
            pltpu.async_copy(
                pos_hbm.at[pl.ds(pos0 + g * pos_ch, pos_ch)], pos_v.at[p],
                psem.at[p])

        def wait_in(p):
            for h in range(2):
                pltpu.make_async_copy(
                    tok_hbm.at[idx_v.at[0, pl.ds(0, H)]],
                    tok_v.at[p, pl.ds(h * H, H)], gsem.at[p]).wait()
            pltpu.make_async_copy(
                pos_hbm.at[pl.ds(0, pos_ch)], pos_v.at[p], psem.at[p]).wait()

        def start_out(g, p):
            pltpu.async_copy(tok_v.at[p, pl.ds(0, H)],
                             out2.at[pl.ds(row0 + g * CH, H)], osem.at[p])
            pltpu.async_copy(tok_v.at[p, pl.ds(H, H)],
                             out2.at[pl.ds(row0 + g * CH + H, H)], osem.at[p])

        def wait_out(p):
            for h in range(2):
                pltpu.make_async_copy(
                    tok_v.at[p, pl.ds(h * H, H)], out2.at[pl.ds(0, H)],
                    osem.at[p]).wait()

        for g in range(N_BUF - 1):
            start_in(g, g)

        def chunk(g, carry):
            p = lax.rem(g, N_BUF)
            wait_in(p)

            def add_col(d, c):
                sl = pl.ds(d * LANES, LANES)
                pvs = [pos_v[p, r, sl] for r in range(pos_ch)]
                for r in range(pos_ch):
                    for b in range(B):
                        plsc.addupdate(tok_v.at[p, r * B + b, sl], pvs[r])
                return c

            lax.fori_loop(0, D // LANES, add_col, 0)

            @pl.when(g + N_BUF - 1 < n_ch)
            def _():
                q = lax.rem(g + N_BUF - 1, N_BUF)

                @pl.when(g >= 1)
                def _():
                    wait_out(q)

                start_in(g + N_BUF - 1, q)

            start_out(g, p)

            return carry

        lax.fori_loop(0, n_ch, chunk, 0)

        for i in range(N_BUF):
            wait_out((n_ch - N_BUF + i) % N_BUF)

    return sc_embed


def kernel(x, token_table, pos_table):
    S, B = x.shape
    D = token_table.shape[1]
    info = plsc.get_sparse_core_info()
    n_workers = info.num_cores * info.num_subcores
    xf = x.reshape(n_workers, -1, CH).astype(jnp.int32)
    return _build_sc_embed(S, B, D, n_workers)(xf, token_table, pos_table)
